# Initial kernel scaffold; baseline (speedup 1.0000x reference)
import jax
import jax.numpy as jnp
from jax.experimental import pallas as pl


def kernel(atom_fea, nbr_fea, nbr_fea_idx, params):
    raise NotImplementedError("write your pallas kernel here")


# speedup vs baseline: 1.3699x; 1.3699x over previous
import functools

import jax
import jax.numpy as jnp
from jax import lax
from jax.experimental import pallas as pl
from jax.experimental.pallas import tpu as pltpu
from jax.experimental.pallas import tpu_sc as plsc

_N = 10000
_M = 32
_A = 128
_NB = 16
_S = 3
_R = _N * _M
_G = 2 * _A * _S

_NW = 32
_RPW = _R // _NW
_CHUNK = 400
_PK = _A // 2


def _sc_gather(table, idx):
    mesh = plsc.VectorSubcoreMesh(core_axis_name="c", subcore_axis_name="s")

    @functools.partial(
        pl.kernel,
        out_type=jax.ShapeDtypeStruct((_R, _PK), jnp.int32),
        mesh=mesh,
        scratch_types=[
            pltpu.VMEM((_CHUNK,), jnp.int32),
            pltpu.VMEM((_CHUNK, _PK), jnp.int32),
            pltpu.SemaphoreType.DMA,
        ],
        compiler_params=pltpu.CompilerParams(use_tc_tiling_on_sc=False),
    )
    def gk(table_hbm, idx_hbm, out_hbm, idx_v, rows_v, sem):
        wid = lax.axis_index("s") * 2 + lax.axis_index("c")
        base = wid * _RPW

        def body(j, carry):
            off = base + j * _CHUNK
            pltpu.sync_copy(idx_hbm.at[pl.ds(off, _CHUNK)], idx_v)
            pltpu.async_copy(table_hbm.at[idx_v], rows_v, sem).wait()
            pltpu.sync_copy(rows_v, out_hbm.at[pl.ds(off, _CHUNK)])
            return carry

        lax.fori_loop(0, _RPW // _CHUNK, body, 0)

    return gk(table, idx)


_BN1 = 80
_RB = _BN1 * _M
_BN3 = 2000


def _sigmoid(x):
    return 1.0 / (1.0 + jnp.exp(-x))


def _softplus(x):
    return jnp.maximum(x, 0.0) + jnp.log1p(jnp.exp(-jnp.abs(x)))


def _p1_body(anbr, e, af, wn, ws, we, sum_o, sq_o):
    g = jnp.dot(anbr[...], wn[...], preferred_element_type=jnp.float32)
    g += jnp.dot(e[...], we[...], preferred_element_type=jnp.float32)
    s = jnp.dot(af[...], ws[...], preferred_element_type=jnp.float32)
    g = g.reshape(_BN1, _M, _G) + s[:, None, :]

    @pl.when(pl.program_id(0) == 0)
    def _():
        sum_o[...] = jnp.zeros_like(sum_o)
        sq_o[...] = jnp.zeros_like(sq_o)

    sum_o[...] += jnp.sum(g, axis=(0, 1))[None]
    sq_o[...] += jnp.sum(g * g, axis=(0, 1))[None]


def _p2_body(anbr, e, af, wn, ws, we, shift, summed_o, s2_o, q2_o):
    g = jnp.dot(anbr[...], wn[...], preferred_element_type=jnp.float32)
    g += jnp.dot(e[...], we[...], preferred_element_type=jnp.float32)
    s = jnp.dot(af[...], ws[...], preferred_element_type=jnp.float32)
    s += shift[...]
    g = g.reshape(_BN1, _M, _G) + s[:, None, :]
    parts = []
    for i in range(_S):
        filt = g[..., i * 2 * _A : i * 2 * _A + _A]
        core = g[..., i * 2 * _A + _A : (i + 1) * 2 * _A]
        parts.append(_sigmoid(filt) * _softplus(core))
    act = jnp.concatenate(parts, axis=-1)
    summed = jnp.sum(act, axis=1)
    summed_o[...] = summed

    @pl.when(pl.program_id(0) == 0)
    def _():
        s2_o[...] = jnp.zeros_like(s2_o)
        q2_o[...] = jnp.zeros_like(q2_o)

    s2_o[...] += jnp.sum(summed, axis=0)[None]
    q2_o[...] += jnp.sum(summed * summed, axis=0)[None]


def _p3_body(summed, af, sc2, sh2, fw, fb, out):
    x = summed[...] * sc2[...] + sh2[...]
    a = af[...]
    af3 = jnp.concatenate([a, a, a], axis=1)
    o = _softplus(af3 + x)
    y = jnp.dot(o, fw[...], preferred_element_type=jnp.float32) + fb[...]
    out[...] = jnp.maximum(y, 0.0)


def _rep_spec(shape):
    return pl.BlockSpec(shape, lambda i: tuple(0 for _ in shape))


def kernel(atom_fea, nbr_fea, nbr_fea_idx, params):
    ws_l, wn_l, we_l, b_l = [], [], [], []
    scales = (1, 2, 3)
    for i, s in enumerate(scales):
        w = params["fc_W_%d" % i]
        b = params["fc_b_%d" % i]
        w_self, w_nbr, w_e = w[:_A], w[_A : 2 * _A], w[2 * _A :]
        if s > 1:
            b = b + params["st_b_%d" % s] @ w_e
            w_e = params["st_W_%d" % s] @ w_e
        ws_l.append(w_self)
        wn_l.append(w_nbr)
        we_l.append(w_e)
        b_l.append(b)
    ws_all = jnp.concatenate(ws_l, axis=1)
    wn_all = jnp.concatenate(wn_l, axis=1)
    we_all = jnp.concatenate(we_l, axis=1)
    b_all = jnp.concatenate(b_l)
    g1 = jnp.concatenate([params["bn1_g_%d" % i] for i in range(_S)])
    b1 = jnp.concatenate([params["bn1_b_%d" % i] for i in range(_S)])
    g2 = jnp.concatenate([params["bn2_g_%d" % i] for i in range(_S)])
    b2 = jnp.concatenate([params["bn2_b_%d" % i] for i in range(_S)])

    table = lax.bitcast_convert_type(
        atom_fea.astype(jnp.bfloat16).reshape(_N, _PK, 2), jnp.int32
    )
    idx_flat = nbr_fea_idx.reshape(-1).astype(jnp.int32)
    packed = _sc_gather(table, idx_flat)
    anbr = lax.bitcast_convert_type(packed, jnp.bfloat16).reshape(_R, _A)

    e2d = nbr_fea.reshape(_R, _NB)
    wn_bf = wn_all.astype(jnp.bfloat16)

    grid1 = _N // _BN1
    sum1, sq1 = pl.pallas_call(
        _p1_body,
        grid=(grid1,),
        in_specs=[
            pl.BlockSpec((_RB, _A), lambda i: (i, 0)),
            pl.BlockSpec((_RB, _NB), lambda i: (i, 0)),
            pl.BlockSpec((_BN1, _A), lambda i: (i, 0)),
            _rep_spec((_A, _G)),
            _rep_spec((_A, _G)),
            _rep_spec((_NB, _G)),
        ],
        out_specs=[_rep_spec((1, _G)), _rep_spec((1, _G))],
        out_shape=[jax.ShapeDtypeStruct((1, _G), jnp.float32)] * 2,
    )(anbr, e2d, atom_fea, wn_bf, ws_all, we_all)

    mu1 = sum1[0] / _R + b_all
    var1 = sq1[0] / _R - (sum1[0] / _R) ** 2
    sc1 = g1 * lax.rsqrt(var1 + 1e-5)
    shift1 = (b_all - mu1) * sc1 + b1

    summed, s2, q2 = pl.pallas_call(
        _p2_body,
        grid=(grid1,),
        in_specs=[
            pl.BlockSpec((_RB, _A), lambda i: (i, 0)),
            pl.BlockSpec((_RB, _NB), lambda i: (i, 0)),
            pl.BlockSpec((_BN1, _A), lambda i: (i, 0)),
            _rep_spec((_A, _G)),
            _rep_spec((_A, _G)),
            _rep_spec((_NB, _G)),
            _rep_spec((1, _G)),
        ],
        out_specs=[
            pl.BlockSpec((_BN1, _S * _A), lambda i: (i, 0)),
            _rep_spec((1, _S * _A)),
            _rep_spec((1, _S * _A)),
        ],
        out_shape=[
            jax.ShapeDtypeStruct((_N, _S * _A), jnp.float32),
            jax.ShapeDtypeStruct((1, _S * _A), jnp.float32),
            jax.ShapeDtypeStruct((1, _S * _A), jnp.float32),
        ],
    )(
        anbr,
        e2d,
        atom_fea,
        (wn_all * sc1).astype(jnp.bfloat16),
        ws_all * sc1,
        we_all * sc1,
        shift1[None],
    )

    mu2 = s2[0] / _N
    var2 = q2[0] / _N - mu2 * mu2
    sc2 = g2 * lax.rsqrt(var2 + 1e-5)
    sh2 = b2 - mu2 * sc2

    out = pl.pallas_call(
        _p3_body,
        grid=(_N // _BN3,),
        in_specs=[
            pl.BlockSpec((_BN3, _S * _A), lambda i: (i, 0)),
            pl.BlockSpec((_BN3, _A), lambda i: (i, 0)),
            _rep_spec((1, _S * _A)),
            _rep_spec((1, _S * _A)),
            _rep_spec((_S * _A, _A)),
            _rep_spec((1, _A)),
        ],
        out_specs=pl.BlockSpec((_BN3, _A), lambda i: (i, 0)),
        out_shape=jax.ShapeDtypeStruct((_N, _A), jnp.float32),
    )(summed, atom_fea, sc2[None], sh2[None], params["fus_W"], params["fus_b"][None])
    return out


# speedup vs baseline: 1.8012x; 1.3148x over previous
import functools

import jax
import jax.numpy as jnp
import numpy as np
from jax import lax
from jax.experimental import pallas as pl
from jax.experimental.pallas import tpu as pltpu
from jax.experimental.pallas import tpu_sc as plsc

_N = 10000
_M = 32
_A = 128
_NB = 16
_S = 3
_R = _N * _M
_G = 2 * _A * _S

_NW = 32
_RPW = _R // _NW
_CHUNK = 400


def _sc_gather(table, idx):
    mesh = plsc.VectorSubcoreMesh(core_axis_name="c", subcore_axis_name="s")

    @functools.partial(
        pl.kernel,
        out_type=jax.ShapeDtypeStruct((_R, _A), jnp.bfloat16),
        mesh=mesh,
        scratch_types=[
            pltpu.VMEM((_CHUNK,), jnp.int32),
            pltpu.VMEM((_CHUNK, _A), jnp.bfloat16),
            pltpu.SemaphoreType.DMA,
        ],
        compiler_params=pltpu.CompilerParams(use_tc_tiling_on_sc=False),
    )
    def gk(table_hbm, idx_hbm, out_hbm, idx_v, rows_v, sem):
        wid = lax.axis_index("s") * 2 + lax.axis_index("c")
        base = wid * _RPW

        def body(j, carry):
            off = base + j * _CHUNK
            pltpu.sync_copy(idx_hbm.at[pl.ds(off, _CHUNK)], idx_v)
            pltpu.async_copy(table_hbm.at[idx_v], rows_v, sem).wait()
            pltpu.sync_copy(rows_v, out_hbm.at[pl.ds(off, _CHUNK)])
            return carry

        lax.fori_loop(0, _RPW // _CHUNK, body, 0)

    return gk(table, idx)


_BN1 = 80
_RB = _BN1 * _M
_BN3 = 2000


def _sigmoid(x):
    return 1.0 / (1.0 + jnp.exp(-x))


def _softplus(x):
    return jnp.maximum(x, 0.0) + jnp.log1p(jnp.exp(-jnp.abs(x)))


def _dot_t(a, b):
    return lax.dot_general(a, b, (((0,), (0,)), ((), ())),
                           preferred_element_type=jnp.float32)


def _p1_body(anbr, e, af, nn_o, ne_o, ns_o, ee_o, es_o, ss_o, cn_o, ce_o, ca_o):
    a = anbr[...]
    ev = e[...]
    fv = af[...]
    a32 = a.astype(jnp.float32)
    nbrsum = jnp.sum(a32.reshape(_BN1, _M, _A), axis=1)
    esum = jnp.sum(ev.reshape(_BN1, _M, _NB), axis=1)

    @pl.when(pl.program_id(0) == 0)
    def _():
        for o in (nn_o, ne_o, ns_o, ee_o, es_o, ss_o, cn_o, ce_o, ca_o):
            o[...] = jnp.zeros_like(o)

    nn_o[...] += _dot_t(a, a)
    ne_o[...] += _dot_t(a, ev)
    ns_o[...] += _dot_t(nbrsum, fv)
    ee_o[...] += _dot_t(ev, ev)
    es_o[...] += _dot_t(esum, fv)
    ss_o[...] += _dot_t(fv, fv)
    cn_o[...] += jnp.sum(nbrsum, axis=0)[None]
    ce_o[...] += jnp.sum(esum, axis=0)[None]
    ca_o[...] += jnp.sum(fv, axis=0)[None]


def _p2_body(anbr, e, af, wn, ws, we, shift, bmat, summed_o, s2_o, q2_o):
    g = jnp.dot(anbr[...], wn[...], preferred_element_type=jnp.float32)
    g += jnp.dot(e[...], we[...], preferred_element_type=jnp.float32)
    s = jnp.dot(af[...], ws[...], preferred_element_type=jnp.float32)
    s += shift[...]
    g += jnp.dot(bmat[...], s.astype(jnp.bfloat16),
                 preferred_element_type=jnp.float32)

    @pl.when(pl.program_id(0) == 0)
    def _():
        s2_o[...] = jnp.zeros_like(s2_o)
        q2_o[...] = jnp.zeros_like(q2_o)

    for i in range(_S):
        filt = g[:, i * 2 * _A : i * 2 * _A + _A]
        core = g[:, i * 2 * _A + _A : (i + 1) * 2 * _A]
        act = _sigmoid(filt) * _softplus(core)
        sm = jnp.sum(act.reshape(_BN1, _M, _A), axis=1)
        summed_o[:, i * _A : (i + 1) * _A] = sm
        s2_o[:, i * _A : (i + 1) * _A] += jnp.sum(sm, axis=0)[None]
        q2_o[:, i * _A : (i + 1) * _A] += jnp.sum(sm * sm, axis=0)[None]


def _p3_body(summed, af, sc2, sh2, fw, fb, out):
    x = summed[...] * sc2[...] + sh2[...]
    a = af[...]
    af3 = jnp.concatenate([a, a, a], axis=1)
    o = _softplus(af3 + x)
    y = jnp.dot(o, fw[...], preferred_element_type=jnp.float32) + fb[...]
    out[...] = jnp.maximum(y, 0.0)


def _rep_spec(shape):
    return pl.BlockSpec(shape, lambda i: tuple(0 for _ in shape))


def kernel(atom_fea, nbr_fea, nbr_fea_idx, params):
    ws_l, wn_l, we_l, b_l = [], [], [], []
    scales = (1, 2, 3)
    for i, s in enumerate(scales):
        w = params["fc_W_%d" % i]
        b = params["fc_b_%d" % i]
        w_self, w_nbr, w_e = w[:_A], w[_A : 2 * _A], w[2 * _A :]
        if s > 1:
            b = b + params["st_b_%d" % s] @ w_e
            w_e = params["st_W_%d" % s] @ w_e
        ws_l.append(w_self)
        wn_l.append(w_nbr)
        we_l.append(w_e)
        b_l.append(b)
    ws_all = jnp.concatenate(ws_l, axis=1)
    wn_all = jnp.concatenate(wn_l, axis=1)
    we_all = jnp.concatenate(we_l, axis=1)
    b_all = jnp.concatenate(b_l)
    g1 = jnp.concatenate([params["bn1_g_%d" % i] for i in range(_S)])
    b1 = jnp.concatenate([params["bn1_b_%d" % i] for i in range(_S)])
    g2 = jnp.concatenate([params["bn2_g_%d" % i] for i in range(_S)])
    b2 = jnp.concatenate([params["bn2_b_%d" % i] for i in range(_S)])

    table = atom_fea.astype(jnp.bfloat16)
    idx_flat = nbr_fea_idx.reshape(-1).astype(jnp.int32)
    anbr = _sc_gather(table, idx_flat)

    e2d = nbr_fea.reshape(_R, _NB)
    e2d_bf = e2d.astype(jnp.bfloat16)

    grid1 = _N // _BN1
    nn, ne, ns, ee, es, ss, cn, ce, ca = pl.pallas_call(
        _p1_body,
        grid=(grid1,),
        in_specs=[
            pl.BlockSpec((_RB, _A), lambda i: (i, 0)),
            pl.BlockSpec((_RB, _NB), lambda i: (i, 0)),
            pl.BlockSpec((_BN1, _A), lambda i: (i, 0)),
        ],
        out_specs=[
            _rep_spec((_A, _A)),
            _rep_spec((_A, _NB)),
            _rep_spec((_A, _A)),
            _rep_spec((_NB, _NB)),
            _rep_spec((_NB, _A)),
            _rep_spec((_A, _A)),
            _rep_spec((1, _A)),
            _rep_spec((1, _NB)),
            _rep_spec((1, _A)),
        ],
        out_shape=[
            jax.ShapeDtypeStruct((_A, _A), jnp.float32),
            jax.ShapeDtypeStruct((_A, _NB), jnp.float32),
            jax.ShapeDtypeStruct((_A, _A), jnp.float32),
            jax.ShapeDtypeStruct((_NB, _NB), jnp.float32),
            jax.ShapeDtypeStruct((_NB, _A), jnp.float32),
            jax.ShapeDtypeStruct((_A, _A), jnp.float32),
            jax.ShapeDtypeStruct((1, _A), jnp.float32),
            jax.ShapeDtypeStruct((1, _NB), jnp.float32),
            jax.ShapeDtypeStruct((1, _A), jnp.float32),
        ],
    )(anbr, e2d_bf, atom_fea)

    c_mat = jnp.concatenate(
        [
            jnp.concatenate([_M * ss, ns.T, es.T], axis=1),
            jnp.concatenate([ns, nn, ne], axis=1),
            jnp.concatenate([es, ne.T, ee], axis=1),
        ],
        axis=0,
    )
    colsum_t = jnp.concatenate([_M * ca[0], cn[0], ce[0]])
    w_full = jnp.concatenate([ws_all, wn_all, we_all], axis=0)
    mean_nb = (colsum_t @ w_full) / _R
    e2_nb = jnp.sum(w_full * (c_mat @ w_full), axis=0) / _R
    var1 = e2_nb - mean_nb * mean_nb
    mu1 = mean_nb + b_all
    sc1 = g1 * lax.rsqrt(var1 + 1e-5)
    shift1 = (b_all - mu1) * sc1 + b1

    bmat = jnp.asarray(
        np.kron(np.eye(_BN1, dtype=np.float32), np.ones((_M, 1), np.float32)),
        dtype=jnp.bfloat16,
    )

    summed, s2, q2 = pl.pallas_call(
        _p2_body,
        grid=(grid1,),
        in_specs=[
            pl.BlockSpec((_RB, _A), lambda i: (i, 0)),
            pl.BlockSpec((_RB, _NB), lambda i: (i, 0)),
            pl.BlockSpec((_BN1, _A), lambda i: (i, 0)),
            _rep_spec((_A, _G)),
            _rep_spec((_A, _G)),
            _rep_spec((_NB, _G)),
            _rep_spec((1, _G)),
            _rep_spec((_RB, _BN1)),
        ],
        out_specs=[
            pl.BlockSpec((_BN1, _S * _A), lambda i: (i, 0)),
            _rep_spec((1, _S * _A)),
            _rep_spec((1, _S * _A)),
        ],
        out_shape=[
            jax.ShapeDtypeStruct((_N, _S * _A), jnp.float32),
            jax.ShapeDtypeStruct((1, _S * _A), jnp.float32),
            jax.ShapeDtypeStruct((1, _S * _A), jnp.float32),
        ],
    )(
        anbr,
        e2d_bf,
        atom_fea,
        (wn_all * sc1).astype(jnp.bfloat16),
        ws_all * sc1,
        (we_all * sc1).astype(jnp.bfloat16),
        shift1[None],
        bmat,
    )

    mu2 = s2[0] / _N
    var2 = q2[0] / _N - mu2 * mu2
    sc2 = g2 * lax.rsqrt(var2 + 1e-5)
    sh2 = b2 - mu2 * sc2

    out = pl.pallas_call(
        _p3_body,
        grid=(_N // _BN3,),
        in_specs=[
            pl.BlockSpec((_BN3, _S * _A), lambda i: (i, 0)),
            pl.BlockSpec((_BN3, _A), lambda i: (i, 0)),
            _rep_spec((1, _S * _A)),
            _rep_spec((1, _S * _A)),
            _rep_spec((_S * _A, _A)),
            _rep_spec((1, _A)),
        ],
        out_specs=pl.BlockSpec((_BN3, _A), lambda i: (i, 0)),
        out_shape=jax.ShapeDtypeStruct((_N, _A), jnp.float32),
    )(summed, atom_fea, sc2[None], sh2[None], params["fus_W"], params["fus_b"][None])
    return out


# speedup vs baseline: 2.1005x; 1.1662x over previous
import functools

import jax
import jax.numpy as jnp
import numpy as np
from jax import lax
from jax.experimental import pallas as pl
from jax.experimental.pallas import tpu as pltpu
from jax.experimental.pallas import tpu_sc as plsc

_N = 10000
_M = 32
_A = 128
_NB = 16
_S = 3
_R = _N * _M
_G = 2 * _A * _S

_NW = 32
_RPW = _R // _NW
_CHUNK = 400


def _sc_gather(table, idx):
    mesh = plsc.VectorSubcoreMesh(core_axis_name="c", subcore_axis_name="s")

    @functools.partial(
        pl.kernel,
        out_type=jax.ShapeDtypeStruct((_R, _A), jnp.bfloat16),
        mesh=mesh,
        scratch_types=[
            pltpu.VMEM((_CHUNK,), jnp.int32),
            pltpu.VMEM((_CHUNK, _A), jnp.bfloat16),
            pltpu.SemaphoreType.DMA,
        ],
        compiler_params=pltpu.CompilerParams(use_tc_tiling_on_sc=False),
    )
    def gk(table_hbm, idx_hbm, out_hbm, idx_v, rows_v, sem):
        wid = lax.axis_index("s") * 2 + lax.axis_index("c")
        base = wid * _RPW

        def body(j, carry):
            off = base + j * _CHUNK
            pltpu.sync_copy(idx_hbm.at[pl.ds(off, _CHUNK)], idx_v)
            pltpu.async_copy(table_hbm.at[idx_v], rows_v, sem).wait()
            pltpu.sync_copy(rows_v, out_hbm.at[pl.ds(off, _CHUNK)])
            return carry

        lax.fori_loop(0, _RPW // _CHUNK, body, 0)

    return gk(table, idx)


_BN1 = 200
_RB = _BN1 * _M
_BN3 = 2000
_BP = 256
_ZP = _BP - _BN1


def _sigmoid(x):
    return 1.0 / (1.0 + jnp.exp(-x))


def _softplus(x):
    return jnp.maximum(x, 0.0) + jnp.log1p(jnp.exp(-jnp.abs(x)))


def _dot_t(a, b):
    return lax.dot_general(a, b, (((0,), (0,)), ((), ())),
                           preferred_element_type=jnp.float32)


def _p1_body(anbr, e, af, nn_o, ne_o, ns_o, ee_o, es_o, ss_o, cn_o, ce_o, ca_o):
    a = anbr[...]
    ev = e[...]
    fv = af[...]
    a32 = a.astype(jnp.float32)
    nbrsum = jnp.sum(a32.reshape(_BN1, _M, _A), axis=1)
    esum = jnp.sum(ev.reshape(_BN1, _M, _NB), axis=1)

    @pl.when(pl.program_id(0) == 0)
    def _():
        for o in (nn_o, ne_o, ns_o, ee_o, es_o, ss_o, cn_o, ce_o, ca_o):
            o[...] = jnp.zeros_like(o)

    nn_o[...] += _dot_t(a, a)
    ne_o[...] += _dot_t(a, ev)
    ns_o[...] += _dot_t(nbrsum, fv)
    ee_o[...] += _dot_t(ev, ev)
    es_o[...] += _dot_t(esum, fv)
    ss_o[...] += _dot_t(fv, fv)
    cn_o[...] += jnp.sum(nbrsum, axis=0)[None]
    ce_o[...] += jnp.sum(esum, axis=0)[None]
    ca_o[...] += jnp.sum(fv, axis=0)[None]


def _p2_body(anbr, e, af, wn, ws, we, shift, bmat, summed_o, s2_o, q2_o):
    s = jnp.dot(af[...], ws[...], preferred_element_type=jnp.float32)
    s += shift[...]
    w2 = jnp.concatenate(
        [wn[...], s.astype(jnp.bfloat16),
         jnp.zeros((_ZP, _G), jnp.bfloat16), we[...]], axis=0)
    x2 = jnp.concatenate([anbr[...], bmat[...], e[...]], axis=1)
    g = jnp.dot(x2, w2, preferred_element_type=jnp.float32)

    @pl.when(pl.program_id(0) == 0)
    def _():
        s2_o[...] = jnp.zeros_like(s2_o)
        q2_o[...] = jnp.zeros_like(q2_o)

    for i in range(_S):
        filt = g[:, i * 2 * _A : i * 2 * _A + _A]
        core = g[:, i * 2 * _A + _A : (i + 1) * 2 * _A]
        act = _sigmoid(filt) * _softplus(core)
        sm = jnp.sum(act.reshape(_BN1, _M, _A), axis=1)
        summed_o[:, i * _A : (i + 1) * _A] = sm
        s2_o[:, i * _A : (i + 1) * _A] += jnp.sum(sm, axis=0)[None]
        q2_o[:, i * _A : (i + 1) * _A] += jnp.sum(sm * sm, axis=0)[None]


def _p3_body(summed, af, sc2, sh2, fw, fb, out):
    x = summed[...] * sc2[...] + sh2[...]
    a = af[...]
    af3 = jnp.concatenate([a, a, a], axis=1)
    o = _softplus(af3 + x)
    y = jnp.dot(o, fw[...], preferred_element_type=jnp.float32) + fb[...]
    out[...] = jnp.maximum(y, 0.0)


def _rep_spec(shape):
    return pl.BlockSpec(shape, lambda i: tuple(0 for _ in shape))


def kernel(atom_fea, nbr_fea, nbr_fea_idx, params):
    ws_l, wn_l, we_l, b_l = [], [], [], []
    scales = (1, 2, 3)
    for i, s in enumerate(scales):
        w = params["fc_W_%d" % i]
        b = params["fc_b_%d" % i]
        w_self, w_nbr, w_e = w[:_A], w[_A : 2 * _A], w[2 * _A :]
        if s > 1:
            b = b + params["st_b_%d" % s] @ w_e
            w_e = params["st_W_%d" % s] @ w_e
        ws_l.append(w_self)
        wn_l.append(w_nbr)
        we_l.append(w_e)
        b_l.append(b)
    ws_all = jnp.concatenate(ws_l, axis=1)
    wn_all = jnp.concatenate(wn_l, axis=1)
    we_all = jnp.concatenate(we_l, axis=1)
    b_all = jnp.concatenate(b_l)
    g1 = jnp.concatenate([params["bn1_g_%d" % i] for i in range(_S)])
    b1 = jnp.concatenate([params["bn1_b_%d" % i] for i in range(_S)])
    g2 = jnp.concatenate([params["bn2_g_%d" % i] for i in range(_S)])
    b2 = jnp.concatenate([params["bn2_b_%d" % i] for i in range(_S)])

    table = atom_fea.astype(jnp.bfloat16)
    idx_flat = nbr_fea_idx.reshape(-1).astype(jnp.int32)
    anbr = _sc_gather(table, idx_flat)

    e2d = nbr_fea.reshape(_R, _NB)
    e2d_bf = e2d.astype(jnp.bfloat16)

    grid1 = _N // _BN1
    nn, ne, ns, ee, es, ss, cn, ce, ca = pl.pallas_call(
        _p1_body,
        grid=(grid1,),
        in_specs=[
            pl.BlockSpec((_RB, _A), lambda i: (i, 0)),
            pl.BlockSpec((_RB, _NB), lambda i: (i, 0)),
            pl.BlockSpec((_BN1, _A), lambda i: (i, 0)),
        ],
        out_specs=[
            _rep_spec((_A, _A)),
            _rep_spec((_A, _NB)),
            _rep_spec((_A, _A)),
            _rep_spec((_NB, _NB)),
            _rep_spec((_NB, _A)),
            _rep_spec((_A, _A)),
            _rep_spec((1, _A)),
            _rep_spec((1, _NB)),
            _rep_spec((1, _A)),
        ],
        out_shape=[
            jax.ShapeDtypeStruct((_A, _A), jnp.float32),
            jax.ShapeDtypeStruct((_A, _NB), jnp.float32),
            jax.ShapeDtypeStruct((_A, _A), jnp.float32),
            jax.ShapeDtypeStruct((_NB, _NB), jnp.float32),
            jax.ShapeDtypeStruct((_NB, _A), jnp.float32),
            jax.ShapeDtypeStruct((_A, _A), jnp.float32),
            jax.ShapeDtypeStruct((1, _A), jnp.float32),
            jax.ShapeDtypeStruct((1, _NB), jnp.float32),
            jax.ShapeDtypeStruct((1, _A), jnp.float32),
        ],
    )(anbr, e2d_bf, atom_fea)

    c_mat = jnp.concatenate(
        [
            jnp.concatenate([_M * ss, ns.T, es.T], axis=1),
            jnp.concatenate([ns, nn, ne], axis=1),
            jnp.concatenate([es, ne.T, ee], axis=1),
        ],
        axis=0,
    )
    colsum_t = jnp.concatenate([_M * ca[0], cn[0], ce[0]])
    w_full = jnp.concatenate([ws_all, wn_all, we_all], axis=0)
    mean_nb = (colsum_t @ w_full) / _R
    e2_nb = jnp.sum(w_full * (c_mat @ w_full), axis=0) / _R
    var1 = e2_nb - mean_nb * mean_nb
    mu1 = mean_nb + b_all
    sc1 = g1 * lax.rsqrt(var1 + 1e-5)
    shift1 = (b_all - mu1) * sc1 + b1

    bmat = jnp.asarray(
        np.pad(
            np.kron(np.eye(_BN1, dtype=np.float32), np.ones((_M, 1), np.float32)),
            ((0, 0), (0, _ZP)),
        ),
        dtype=jnp.bfloat16,
    )

    summed, s2, q2 = pl.pallas_call(
        _p2_body,
        grid=(grid1,),
        in_specs=[
            pl.BlockSpec((_RB, _A), lambda i: (i, 0)),
            pl.BlockSpec((_RB, _NB), lambda i: (i, 0)),
            pl.BlockSpec((_BN1, _A), lambda i: (i, 0)),
            _rep_spec((_A, _G)),
            _rep_spec((_A, _G)),
            _rep_spec((_NB, _G)),
            _rep_spec((1, _G)),
            _rep_spec((_RB, _BP)),
        ],
        out_specs=[
            pl.BlockSpec((_BN1, _S * _A), lambda i: (i, 0)),
            _rep_spec((1, _S * _A)),
            _rep_spec((1, _S * _A)),
        ],
        out_shape=[
            jax.ShapeDtypeStruct((_N, _S * _A), jnp.float32),
            jax.ShapeDtypeStruct((1, _S * _A), jnp.float32),
            jax.ShapeDtypeStruct((1, _S * _A), jnp.float32),
        ],
    )(
        anbr,
        e2d_bf,
        atom_fea,
        (wn_all * sc1).astype(jnp.bfloat16),
        ws_all * sc1,
        (we_all * sc1).astype(jnp.bfloat16),
        shift1[None],
        bmat,
    )

    mu2 = s2[0] / _N
    var2 = q2[0] / _N - mu2 * mu2
    sc2 = g2 * lax.rsqrt(var2 + 1e-5)
    sh2 = b2 - mu2 * sc2

    out = pl.pallas_call(
        _p3_body,
        grid=(_N // _BN3,),
        in_specs=[
            pl.BlockSpec((_BN3, _S * _A), lambda i: (i, 0)),
            pl.BlockSpec((_BN3, _A), lambda i: (i, 0)),
            _rep_spec((1, _S * _A)),
            _rep_spec((1, _S * _A)),
            _rep_spec((_S * _A, _A)),
            _rep_spec((1, _A)),
        ],
        out_specs=pl.BlockSpec((_BN3, _A), lambda i: (i, 0)),
        out_shape=jax.ShapeDtypeStruct((_N, _A), jnp.float32),
    )(summed, atom_fea, sc2[None], sh2[None], params["fus_W"], params["fus_b"][None])
    return out


# speedup vs baseline: 2.4812x; 1.1813x over previous
import functools

import jax
import jax.numpy as jnp
import numpy as np
from jax import lax
from jax.experimental import pallas as pl
from jax.experimental.pallas import tpu as pltpu
from jax.experimental.pallas import tpu_sc as plsc

_N = 10000
_M = 32
_A = 128
_NB = 16
_S = 3
_R = _N * _M
_G = 2 * _A * _S

_NW = 32
_RPW = _R // _NW
_CHUNK = 400


def _sc_gather(table, idx):
    mesh = plsc.VectorSubcoreMesh(core_axis_name="c", subcore_axis_name="s")

    @functools.partial(
        pl.kernel,
        out_type=jax.ShapeDtypeStruct((_R, _A), jnp.bfloat16),
        mesh=mesh,
        scratch_types=[
            pltpu.VMEM((_CHUNK,), jnp.int32),
            pltpu.VMEM((_CHUNK, _A), jnp.bfloat16),
            pltpu.SemaphoreType.DMA,
        ],
        compiler_params=pltpu.CompilerParams(use_tc_tiling_on_sc=False),
    )
    def gk(table_hbm, idx_hbm, out_hbm, idx_v, rows_v, sem):
        wid = lax.axis_index("s") * 2 + lax.axis_index("c")
        base = wid * _RPW

        def body(j, carry):
            off = base + j * _CHUNK
            pltpu.sync_copy(idx_hbm.at[pl.ds(off, _CHUNK)], idx_v)
            pltpu.async_copy(table_hbm.at[idx_v], rows_v, sem).wait()
            pltpu.sync_copy(rows_v, out_hbm.at[pl.ds(off, _CHUNK)])
            return carry

        lax.fori_loop(0, _RPW // _CHUNK, body, 0)

    return gk(table, idx)


_BN1 = 200
_RB = _BN1 * _M
_BN3 = 2000
_BP = 256
_ZP = _BP - _BN1


_LOG2E = 1.4426950408889634
_LN2 = 0.6931471805599453


def _sigmoid(x):
    return 1.0 / (1.0 + jnp.exp2(x * -_LOG2E))


def _softplus(x):
    u = jnp.exp2(jnp.abs(x) * -_LOG2E)
    return jnp.maximum(x, 0.0) + _LN2 * jnp.log2(1.0 + u)


def _dot_t(a, b):
    return lax.dot_general(a, b, (((0,), (0,)), ((), ())),
                           preferred_element_type=jnp.float32)


def _p1_body(anbr, e, af, nn_o, ne_o, ns_o, ee_o, es_o, ss_o, cn_o, ce_o, ca_o):
    a = anbr[...]
    ev = e[...]
    fv = af[...]
    a32 = a.astype(jnp.float32)
    nbrsum = jnp.sum(a32.reshape(_BN1, _M, _A), axis=1)
    esum = jnp.sum(ev.reshape(_BN1, _M, _NB), axis=1)

    @pl.when(pl.program_id(0) == 0)
    def _():
        for o in (nn_o, ne_o, ns_o, ee_o, es_o, ss_o, cn_o, ce_o, ca_o):
            o[...] = jnp.zeros_like(o)

    nn_o[...] += _dot_t(a, a)
    ne_o[...] += _dot_t(a, ev)
    ns_o[...] += _dot_t(nbrsum, fv)
    ee_o[...] += _dot_t(ev, ev)
    es_o[...] += _dot_t(esum, fv)
    ss_o[...] += _dot_t(fv, fv)
    cn_o[...] += jnp.sum(nbrsum, axis=0)[None]
    ce_o[...] += jnp.sum(esum, axis=0)[None]
    ca_o[...] += jnp.sum(fv, axis=0)[None]


def _p2_body(anbr, e, af, wn, ws, we, shift, bmat, summed_o, s2_o, q2_o):
    s = jnp.dot(af[...], ws[...], preferred_element_type=jnp.float32)
    s += shift[...]
    w2 = jnp.concatenate(
        [wn[...], s.astype(jnp.bfloat16),
         jnp.zeros((_ZP, _G), jnp.bfloat16), we[...]], axis=0)
    x2 = jnp.concatenate([anbr[...], bmat[...], e[...]], axis=1)
    g = jnp.dot(x2, w2, preferred_element_type=jnp.float32)

    @pl.when(pl.program_id(0) == 0)
    def _():
        s2_o[...] = jnp.zeros_like(s2_o)
        q2_o[...] = jnp.zeros_like(q2_o)

    for i in range(_S):
        filt = g[:, i * 2 * _A : i * 2 * _A + _A]
        core = g[:, i * 2 * _A + _A : (i + 1) * 2 * _A]
        act = (_sigmoid(filt) * _softplus(core)).astype(jnp.bfloat16)
        sm = _dot_t(bmat[...], act)[:_BN1]
        summed_o[:, i * _A : (i + 1) * _A] = sm
        s2_o[:, i * _A : (i + 1) * _A] += jnp.sum(sm, axis=0)[None]
        q2_o[:, i * _A : (i + 1) * _A] += jnp.sum(sm * sm, axis=0)[None]


def _p3_body(summed, af, sc2, sh2, fw, fb, out):
    x = summed[...] * sc2[...] + sh2[...]
    a = af[...]
    af3 = jnp.concatenate([a, a, a], axis=1)
    o = _softplus(af3 + x)
    y = jnp.dot(o, fw[...], preferred_element_type=jnp.float32) + fb[...]
    out[...] = jnp.maximum(y, 0.0)


def _rep_spec(shape):
    return pl.BlockSpec(shape, lambda i: tuple(0 for _ in shape))


def kernel(atom_fea, nbr_fea, nbr_fea_idx, params):
    ws_l, wn_l, we_l, b_l = [], [], [], []
    scales = (1, 2, 3)
    for i, s in enumerate(scales):
        w = params["fc_W_%d" % i]
        b = params["fc_b_%d" % i]
        w_self, w_nbr, w_e = w[:_A], w[_A : 2 * _A], w[2 * _A :]
        if s > 1:
            b = b + params["st_b_%d" % s] @ w_e
            w_e = params["st_W_%d" % s] @ w_e
        ws_l.append(w_self)
        wn_l.append(w_nbr)
        we_l.append(w_e)
        b_l.append(b)
    ws_all = jnp.concatenate(ws_l, axis=1)
    wn_all = jnp.concatenate(wn_l, axis=1)
    we_all = jnp.concatenate(we_l, axis=1)
    b_all = jnp.concatenate(b_l)
    g1 = jnp.concatenate([params["bn1_g_%d" % i] for i in range(_S)])
    b1 = jnp.concatenate([params["bn1_b_%d" % i] for i in range(_S)])
    g2 = jnp.concatenate([params["bn2_g_%d" % i] for i in range(_S)])
    b2 = jnp.concatenate([params["bn2_b_%d" % i] for i in range(_S)])

    table = atom_fea.astype(jnp.bfloat16)
    idx_flat = nbr_fea_idx.reshape(-1).astype(jnp.int32)
    anbr = _sc_gather(table, idx_flat)

    e2d = nbr_fea.reshape(_R, _NB)
    e2d_bf = e2d.astype(jnp.bfloat16)

    grid1 = _N // _BN1
    nn, ne, ns, ee, es, ss, cn, ce, ca = pl.pallas_call(
        _p1_body,
        grid=(grid1,),
        in_specs=[
            pl.BlockSpec((_RB, _A), lambda i: (i, 0)),
            pl.BlockSpec((_RB, _NB), lambda i: (i, 0)),
            pl.BlockSpec((_BN1, _A), lambda i: (i, 0)),
        ],
        out_specs=[
            _rep_spec((_A, _A)),
            _rep_spec((_A, _NB)),
            _rep_spec((_A, _A)),
            _rep_spec((_NB, _NB)),
            _rep_spec((_NB, _A)),
            _rep_spec((_A, _A)),
            _rep_spec((1, _A)),
            _rep_spec((1, _NB)),
            _rep_spec((1, _A)),
        ],
        out_shape=[
            jax.ShapeDtypeStruct((_A, _A), jnp.float32),
            jax.ShapeDtypeStruct((_A, _NB), jnp.float32),
            jax.ShapeDtypeStruct((_A, _A), jnp.float32),
            jax.ShapeDtypeStruct((_NB, _NB), jnp.float32),
            jax.ShapeDtypeStruct((_NB, _A), jnp.float32),
            jax.ShapeDtypeStruct((_A, _A), jnp.float32),
            jax.ShapeDtypeStruct((1, _A), jnp.float32),
            jax.ShapeDtypeStruct((1, _NB), jnp.float32),
            jax.ShapeDtypeStruct((1, _A), jnp.float32),
        ],
    )(anbr, e2d_bf, atom_fea)

    c_mat = jnp.concatenate(
        [
            jnp.concatenate([_M * ss, ns.T, es.T], axis=1),
            jnp.concatenate([ns, nn, ne], axis=1),
            jnp.concatenate([es, ne.T, ee], axis=1),
        ],
        axis=0,
    )
    colsum_t = jnp.concatenate([_M * ca[0], cn[0], ce[0]])
    w_full = jnp.concatenate([ws_all, wn_all, we_all], axis=0)
    mean_nb = (colsum_t @ w_full) / _R
    e2_nb = jnp.sum(w_full * (c_mat @ w_full), axis=0) / _R
    var1 = e2_nb - mean_nb * mean_nb
    mu1 = mean_nb + b_all
    sc1 = g1 * lax.rsqrt(var1 + 1e-5)
    shift1 = (b_all - mu1) * sc1 + b1

    bmat = jnp.asarray(
        np.pad(
            np.kron(np.eye(_BN1, dtype=np.float32), np.ones((_M, 1), np.float32)),
            ((0, 0), (0, _ZP)),
        ),
        dtype=jnp.bfloat16,
    )

    summed, s2, q2 = pl.pallas_call(
        _p2_body,
        grid=(grid1,),
        in_specs=[
            pl.BlockSpec((_RB, _A), lambda i: (i, 0)),
            pl.BlockSpec((_RB, _NB), lambda i: (i, 0)),
            pl.BlockSpec((_BN1, _A), lambda i: (i, 0)),
            _rep_spec((_A, _G)),
            _rep_spec((_A, _G)),
            _rep_spec((_NB, _G)),
            _rep_spec((1, _G)),
            _rep_spec((_RB, _BP)),
        ],
        out_specs=[
            pl.BlockSpec((_BN1, _S * _A), lambda i: (i, 0)),
            _rep_spec((1, _S * _A)),
            _rep_spec((1, _S * _A)),
        ],
        out_shape=[
            jax.ShapeDtypeStruct((_N, _S * _A), jnp.float32),
            jax.ShapeDtypeStruct((1, _S * _A), jnp.float32),
            jax.ShapeDtypeStruct((1, _S * _A), jnp.float32),
        ],
    )(
        anbr,
        e2d_bf,
        atom_fea,
        (wn_all * sc1).astype(jnp.bfloat16),
        ws_all * sc1,
        (we_all * sc1).astype(jnp.bfloat16),
        shift1[None],
        bmat,
    )

    mu2 = s2[0] / _N
    var2 = q2[0] / _N - mu2 * mu2
    sc2 = g2 * lax.rsqrt(var2 + 1e-5)
    sh2 = b2 - mu2 * sc2

    out = pl.pallas_call(
        _p3_body,
        grid=(_N // _BN3,),
        in_specs=[
            pl.BlockSpec((_BN3, _S * _A), lambda i: (i, 0)),
            pl.BlockSpec((_BN3, _A), lambda i: (i, 0)),
            _rep_spec((1, _S * _A)),
            _rep_spec((1, _S * _A)),
            _rep_spec((_S * _A, _A)),
            _rep_spec((1, _A)),
        ],
        out_specs=pl.BlockSpec((_BN3, _A), lambda i: (i, 0)),
        out_shape=jax.ShapeDtypeStruct((_N, _A), jnp.float32),
    )(summed, atom_fea, sc2[None], sh2[None], params["fus_W"], params["fus_b"][None])
    return out


# speedup vs baseline: 2.5247x; 1.0175x over previous
import functools

import jax
import jax.numpy as jnp
import numpy as np
from jax import lax
from jax.experimental import pallas as pl
from jax.experimental.pallas import tpu as pltpu
from jax.experimental.pallas import tpu_sc as plsc

_N = 10000
_M = 32
_A = 128
_NB = 16
_S = 3
_R = _N * _M
_G = 2 * _A * _S

_NW = 32
_RPW = _R // _NW
_CHUNK = 200


def _sc_gather(table, idx):
    mesh = plsc.VectorSubcoreMesh(core_axis_name="c", subcore_axis_name="s")

    @functools.partial(
        pl.kernel,
        out_type=jax.ShapeDtypeStruct((_R, _A), jnp.bfloat16),
        mesh=mesh,
        scratch_types=[
            pltpu.VMEM((_RPW,), jnp.int32),
            pltpu.VMEM((2, _CHUNK, _A), jnp.bfloat16),
            pltpu.SemaphoreType.DMA,
            pltpu.SemaphoreType.DMA,
        ],
        compiler_params=pltpu.CompilerParams(use_tc_tiling_on_sc=False),
    )
    def gk(table_hbm, idx_hbm, out_hbm, idx_v, rows_v, sem0, sem1):
        wid = lax.axis_index("s") * 2 + lax.axis_index("c")
        base = wid * _RPW
        nchunk = _RPW // _CHUNK
        sems = (sem0, sem1)
        pltpu.sync_copy(idx_hbm.at[pl.ds(base, _RPW)], idx_v)

        def start(j, b):
            pltpu.async_copy(
                table_hbm.at[idx_v.at[pl.ds(j * _CHUNK, _CHUNK)]],
                rows_v.at[b],
                sems[b],
            )

        def finish(j, b):
            pltpu.make_async_copy(
                table_hbm.at[idx_v.at[pl.ds(0, _CHUNK)]],
                rows_v.at[b],
                sems[b],
            ).wait()
            pltpu.sync_copy(
                rows_v.at[b], out_hbm.at[pl.ds(base + j * _CHUNK, _CHUNK)]
            )

        start(0, 0)

        def body(p, carry):
            j0 = p * 2
            start(j0 + 1, 1)
            finish(j0, 0)

            @pl.when(j0 + 2 < nchunk)
            def _():
                start(j0 + 2, 0)

            finish(j0 + 1, 1)
            return carry

        lax.fori_loop(0, nchunk // 2, body, 0)

    return gk(table, idx)


_BN1 = 200
_RB = _BN1 * _M
_BN3 = 2000
_BP = 256
_ZP = _BP - _BN1


_LOG2E = 1.4426950408889634
_LN2 = 0.6931471805599453


def _sigmoid(x):
    return 1.0 / (1.0 + jnp.exp2(x * -_LOG2E))


def _softplus(x):
    u = jnp.exp2(jnp.abs(x) * -_LOG2E)
    return jnp.maximum(x, 0.0) + _LN2 * jnp.log2(1.0 + u)


def _dot_t(a, b):
    return lax.dot_general(a, b, (((0,), (0,)), ((), ())),
                           preferred_element_type=jnp.float32)


def _p1_body(anbr, e, af, nn_o, ne_o, ns_o, ee_o, es_o, ss_o, cn_o, ce_o, ca_o):
    a = anbr[...]
    ev = e[...]
    fv = af[...]
    a32 = a.astype(jnp.float32)
    nbrsum = jnp.sum(a32.reshape(_BN1, _M, _A), axis=1)
    esum = jnp.sum(ev.reshape(_BN1, _M, _NB), axis=1)

    @pl.when(pl.program_id(0) == 0)
    def _():
        for o in (nn_o, ne_o, ns_o, ee_o, es_o, ss_o, cn_o, ce_o, ca_o):
            o[...] = jnp.zeros_like(o)

    nn_o[...] += _dot_t(a, a)
    ne_o[...] += _dot_t(a, ev)
    ns_o[...] += _dot_t(nbrsum, fv)
    ee_o[...] += _dot_t(ev, ev)
    es_o[...] += _dot_t(esum, fv)
    ss_o[...] += _dot_t(fv, fv)
    cn_o[...] += jnp.sum(nbrsum, axis=0)[None]
    ce_o[...] += jnp.sum(esum, axis=0)[None]
    ca_o[...] += jnp.sum(fv, axis=0)[None]


def _p2_body(anbr, e, af, wn, ws, we, shift, bmat, bmat_t, summed_o, s2_o, q2_o):
    s = jnp.dot(af[...], ws[...], preferred_element_type=jnp.float32)
    s += shift[...]
    w2 = jnp.concatenate(
        [wn[...], s.astype(jnp.bfloat16),
         jnp.zeros((_ZP, _G), jnp.bfloat16), we[...]], axis=0)
    x2 = jnp.concatenate([anbr[...], bmat[...], e[...]], axis=1)
    g = jnp.dot(x2, w2, preferred_element_type=jnp.float32)

    @pl.when(pl.program_id(0) == 0)
    def _():
        s2_o[...] = jnp.zeros_like(s2_o)
        q2_o[...] = jnp.zeros_like(q2_o)

    for i in range(_S):
        filt = g[:, i * 2 * _A : i * 2 * _A + _A]
        core = g[:, i * 2 * _A + _A : (i + 1) * 2 * _A]
        act = (_sigmoid(filt) * _softplus(core)).astype(jnp.bfloat16)
        sm = jnp.dot(bmat_t[...], act,
                     preferred_element_type=jnp.float32)
        summed_o[:, i * _A : (i + 1) * _A] = sm
        s2_o[:, i * _A : (i + 1) * _A] += jnp.sum(sm, axis=0)[None]
        q2_o[:, i * _A : (i + 1) * _A] += jnp.sum(sm * sm, axis=0)[None]


def _p3_body(summed, af, sc2, sh2, fw, fb, out):
    x = summed[...] * sc2[...] + sh2[...]
    a = af[...]
    af3 = jnp.concatenate([a, a, a], axis=1)
    o = _softplus(af3 + x)
    y = jnp.dot(o, fw[...], preferred_element_type=jnp.float32) + fb[...]
    out[...] = jnp.maximum(y, 0.0)


def _rep_spec(shape):
    return pl.BlockSpec(shape, lambda i: tuple(0 for _ in shape))


def kernel(atom_fea, nbr_fea, nbr_fea_idx, params):
    ws_l, wn_l, we_l, b_l = [], [], [], []
    scales = (1, 2, 3)
    for i, s in enumerate(scales):
        w = params["fc_W_%d" % i]
        b = params["fc_b_%d" % i]
        w_self, w_nbr, w_e = w[:_A], w[_A : 2 * _A], w[2 * _A :]
        if s > 1:
            b = b + params["st_b_%d" % s] @ w_e
            w_e = params["st_W_%d" % s] @ w_e
        ws_l.append(w_self)
        wn_l.append(w_nbr)
        we_l.append(w_e)
        b_l.append(b)
    ws_all = jnp.concatenate(ws_l, axis=1)
    wn_all = jnp.concatenate(wn_l, axis=1)
    we_all = jnp.concatenate(we_l, axis=1)
    b_all = jnp.concatenate(b_l)
    g1 = jnp.concatenate([params["bn1_g_%d" % i] for i in range(_S)])
    b1 = jnp.concatenate([params["bn1_b_%d" % i] for i in range(_S)])
    g2 = jnp.concatenate([params["bn2_g_%d" % i] for i in range(_S)])
    b2 = jnp.concatenate([params["bn2_b_%d" % i] for i in range(_S)])

    table = atom_fea.astype(jnp.bfloat16)
    idx_flat = nbr_fea_idx.reshape(-1).astype(jnp.int32)
    anbr = _sc_gather(table, idx_flat)

    e2d = nbr_fea.reshape(_R, _NB)
    e2d_bf = e2d.astype(jnp.bfloat16)

    grid1 = _N // _BN1
    nn, ne, ns, ee, es, ss, cn, ce, ca = pl.pallas_call(
        _p1_body,
        grid=(grid1,),
        in_specs=[
            pl.BlockSpec((_RB, _A), lambda i: (i, 0)),
            pl.BlockSpec((_RB, _NB), lambda i: (i, 0)),
            pl.BlockSpec((_BN1, _A), lambda i: (i, 0)),
        ],
        out_specs=[
            _rep_spec((_A, _A)),
            _rep_spec((_A, _NB)),
            _rep_spec((_A, _A)),
            _rep_spec((_NB, _NB)),
            _rep_spec((_NB, _A)),
            _rep_spec((_A, _A)),
            _rep_spec((1, _A)),
            _rep_spec((1, _NB)),
            _rep_spec((1, _A)),
        ],
        out_shape=[
            jax.ShapeDtypeStruct((_A, _A), jnp.float32),
            jax.ShapeDtypeStruct((_A, _NB), jnp.float32),
            jax.ShapeDtypeStruct((_A, _A), jnp.float32),
            jax.ShapeDtypeStruct((_NB, _NB), jnp.float32),
            jax.ShapeDtypeStruct((_NB, _A), jnp.float32),
            jax.ShapeDtypeStruct((_A, _A), jnp.float32),
            jax.ShapeDtypeStruct((1, _A), jnp.float32),
            jax.ShapeDtypeStruct((1, _NB), jnp.float32),
            jax.ShapeDtypeStruct((1, _A), jnp.float32),
        ],
    )(anbr, e2d_bf, atom_fea)

    c_mat = jnp.concatenate(
        [
            jnp.concatenate([_M * ss, ns.T, es.T], axis=1),
            jnp.concatenate([ns, nn, ne], axis=1),
            jnp.concatenate([es, ne.T, ee], axis=1),
        ],
        axis=0,
    )
    colsum_t = jnp.concatenate([_M * ca[0], cn[0], ce[0]])
    w_full = jnp.concatenate([ws_all, wn_all, we_all], axis=0)
    mean_nb = (colsum_t @ w_full) / _R
    e2_nb = jnp.sum(w_full * (c_mat @ w_full), axis=0) / _R
    var1 = e2_nb - mean_nb * mean_nb
    mu1 = mean_nb + b_all
    sc1 = g1 * lax.rsqrt(var1 + 1e-5)
    shift1 = (b_all - mu1) * sc1 + b1

    bmat = jnp.asarray(
        np.pad(
            np.kron(np.eye(_BN1, dtype=np.float32), np.ones((_M, 1), np.float32)),
            ((0, 0), (0, _ZP)),
        ),
        dtype=jnp.bfloat16,
    )
    bmat_t = jnp.asarray(
        np.kron(np.eye(_BN1, dtype=np.float32), np.ones((1, _M), np.float32)),
        dtype=jnp.bfloat16,
    )

    summed, s2, q2 = pl.pallas_call(
        _p2_body,
        grid=(grid1,),
        in_specs=[
            pl.BlockSpec((_RB, _A), lambda i: (i, 0)),
            pl.BlockSpec((_RB, _NB), lambda i: (i, 0)),
            pl.BlockSpec((_BN1, _A), lambda i: (i, 0)),
            _rep_spec((_A, _G)),
            _rep_spec((_A, _G)),
            _rep_spec((_NB, _G)),
            _rep_spec((1, _G)),
            _rep_spec((_RB, _BP)),
            _rep_spec((_BN1, _RB)),
        ],
        out_specs=[
            pl.BlockSpec((_BN1, _S * _A), lambda i: (i, 0)),
            _rep_spec((1, _S * _A)),
            _rep_spec((1, _S * _A)),
        ],
        out_shape=[
            jax.ShapeDtypeStruct((_N, _S * _A), jnp.float32),
            jax.ShapeDtypeStruct((1, _S * _A), jnp.float32),
            jax.ShapeDtypeStruct((1, _S * _A), jnp.float32),
        ],
    )(
        anbr,
        e2d_bf,
        atom_fea,
        (wn_all * sc1).astype(jnp.bfloat16),
        ws_all * sc1,
        (we_all * sc1).astype(jnp.bfloat16),
        shift1[None],
        bmat,
        bmat_t,
    )

    mu2 = s2[0] / _N
    var2 = q2[0] / _N - mu2 * mu2
    sc2 = g2 * lax.rsqrt(var2 + 1e-5)
    sh2 = b2 - mu2 * sc2

    out = pl.pallas_call(
        _p3_body,
        grid=(_N // _BN3,),
        in_specs=[
            pl.BlockSpec((_BN3, _S * _A), lambda i: (i, 0)),
            pl.BlockSpec((_BN3, _A), lambda i: (i, 0)),
            _rep_spec((1, _S * _A)),
            _rep_spec((1, _S * _A)),
            _rep_spec((_S * _A, _A)),
            _rep_spec((1, _A)),
        ],
        out_specs=pl.BlockSpec((_BN3, _A), lambda i: (i, 0)),
        out_shape=jax.ShapeDtypeStruct((_N, _A), jnp.float32),
    )(summed, atom_fea, sc2[None], sh2[None], params["fus_W"], params["fus_b"][None])
    return out


# speedup vs baseline: 2.5408x; 1.0064x over previous
import functools

import jax
import jax.numpy as jnp
import numpy as np
from jax import lax
from jax.experimental import pallas as pl
from jax.experimental.pallas import tpu as pltpu
from jax.experimental.pallas import tpu_sc as plsc

_N = 10000
_M = 32
_A = 128
_NB = 16
_S = 3
_R = _N * _M
_G = 2 * _A * _S

_NW = 32
_RPW = _R // _NW
_CHUNK = 200


def _sc_gather(table, idx):
    mesh = plsc.VectorSubcoreMesh(core_axis_name="c", subcore_axis_name="s")

    @functools.partial(
        pl.kernel,
        out_type=jax.ShapeDtypeStruct((_R, _A), jnp.bfloat16),
        mesh=mesh,
        scratch_types=[
            pltpu.VMEM((_RPW,), jnp.int32),
            pltpu.VMEM((2, _CHUNK, _A), jnp.bfloat16),
            pltpu.SemaphoreType.DMA,
            pltpu.SemaphoreType.DMA,
        ],
        compiler_params=pltpu.CompilerParams(use_tc_tiling_on_sc=False),
    )
    def gk(table_hbm, idx_hbm, out_hbm, idx_v, rows_v, sem0, sem1):
        wid = lax.axis_index("s") * 2 + lax.axis_index("c")
        base = wid * _RPW
        nchunk = _RPW // _CHUNK
        sems = (sem0, sem1)
        pltpu.sync_copy(idx_hbm.at[pl.ds(base, _RPW)], idx_v)

        def start(j, b):
            pltpu.async_copy(
                table_hbm.at[idx_v.at[pl.ds(j * _CHUNK, _CHUNK)]],
                rows_v.at[b],
                sems[b],
            )

        def finish(j, b):
            pltpu.make_async_copy(
                table_hbm.at[idx_v.at[pl.ds(0, _CHUNK)]],
                rows_v.at[b],
                sems[b],
            ).wait()
            pltpu.sync_copy(
                rows_v.at[b], out_hbm.at[pl.ds(base + j * _CHUNK, _CHUNK)]
            )

        start(0, 0)

        def body(p, carry):
            j0 = p * 2
            start(j0 + 1, 1)
            finish(j0, 0)

            @pl.when(j0 + 2 < nchunk)
            def _():
                start(j0 + 2, 0)

            finish(j0 + 1, 1)
            return carry

        lax.fori_loop(0, nchunk // 2, body, 0)

    return gk(table, idx)


_BN1 = 200
_RB = _BN1 * _M
_BN3 = 2000
_BP = 256
_ZP = _BP - _BN1


_LOG2E = 1.4426950408889634
_LN2 = 0.6931471805599453


def _sigmoid(x):
    return 1.0 / (1.0 + jnp.exp2(x * -_LOG2E))


def _softplus(x):
    u = jnp.exp2(jnp.abs(x) * -_LOG2E)
    return jnp.maximum(x, 0.0) + _LN2 * jnp.log2(1.0 + u)


def _dot_t(a, b):
    return lax.dot_general(a, b, (((0,), (0,)), ((), ())),
                           preferred_element_type=jnp.float32)


def _p1_body(anbr, e, af, nn_o, ne_o, ns_o, ee_o, es_o, ss_o, cn_o, ce_o, ca_o):
    a = anbr[...]
    ev = e[...]
    fv = af[...]
    a32 = a.astype(jnp.float32)
    nbrsum = jnp.sum(a32.reshape(_BN1, _M, _A), axis=1)
    esum = jnp.sum(ev.reshape(_BN1, _M, _NB), axis=1)

    @pl.when(pl.program_id(0) == 0)
    def _():
        for o in (nn_o, ne_o, ns_o, ee_o, es_o, ss_o, cn_o, ce_o, ca_o):
            o[...] = jnp.zeros_like(o)

    nn_o[...] += _dot_t(a, a)
    ne_o[...] += _dot_t(a, ev)
    ns_o[...] += _dot_t(nbrsum, fv)
    ee_o[...] += _dot_t(ev, ev)
    es_o[...] += _dot_t(esum, fv)
    ss_o[...] += _dot_t(fv, fv)
    cn_o[...] += jnp.sum(nbrsum, axis=0)[None]
    ce_o[...] += jnp.sum(esum, axis=0)[None]
    ca_o[...] += jnp.sum(fv, axis=0)[None]


def _fdot(a, b):
    return jnp.dot(a, b, preferred_element_type=jnp.float32)


def _p2_body(anbr, e, af, wn, ws, we, g1, b1, nn, ne, ns, ee, es, ss,
             cn, ce, ca, bmat, bmat_t, summed_o, s2_o, q2_o,
             wn2_s, ws2_s, we2_s, shift_s):
    @pl.when(pl.program_id(0) == 0)
    def _():
        wnv, wsv, wev = wn[...], ws[...], we[...]
        mean = (_M * _fdot(ca[...], wsv) + _fdot(cn[...], wnv)
                + _fdot(ce[...], wev)) / _R
        e2 = (
            _M * jnp.sum(wsv * _fdot(ss[...], wsv), axis=0)
            + jnp.sum(wnv * _fdot(nn[...], wnv), axis=0)
            + jnp.sum(wev * _fdot(ee[...], wev), axis=0)
            + 2.0 * jnp.sum(wnv * _fdot(ns[...], wsv), axis=0)
            + 2.0 * jnp.sum(wev * _fdot(es[...], wsv), axis=0)
            + 2.0 * jnp.sum(wnv * _fdot(ne[...], wev), axis=0)
        )[None] / _R
        var = e2 - mean * mean
        sc1 = g1[...] * lax.rsqrt(var + 1e-5)
        shift_s[...] = b1[...] - mean * sc1
        wn2_s[...] = (wnv * sc1).astype(jnp.bfloat16)
        ws2_s[...] = wsv * sc1
        we2_s[...] = (wev * sc1).astype(jnp.bfloat16)

    s = jnp.dot(af[...], ws2_s[...], preferred_element_type=jnp.float32)
    s += shift_s[...]
    w2 = jnp.concatenate(
        [wn2_s[...], s.astype(jnp.bfloat16),
         jnp.zeros((_ZP, _G), jnp.bfloat16), we2_s[...]], axis=0)
    x2 = jnp.concatenate([anbr[...], bmat[...], e[...]], axis=1)
    g = jnp.dot(x2, w2, preferred_element_type=jnp.float32)

    @pl.when(pl.program_id(0) == 0)
    def _():
        s2_o[...] = jnp.zeros_like(s2_o)
        q2_o[...] = jnp.zeros_like(q2_o)

    for i in range(_S):
        filt = g[:, i * 2 * _A : i * 2 * _A + _A]
        core = g[:, i * 2 * _A + _A : (i + 1) * 2 * _A]
        act = (_sigmoid(filt) * _softplus(core)).astype(jnp.bfloat16)
        sm = jnp.dot(bmat_t[...], act,
                     preferred_element_type=jnp.float32)
        summed_o[:, i * _A : (i + 1) * _A] = sm
        s2_o[:, i * _A : (i + 1) * _A] += jnp.sum(sm, axis=0)[None]
        q2_o[:, i * _A : (i + 1) * _A] += jnp.sum(sm * sm, axis=0)[None]


def _p3_body(summed, af, s2, q2, g2, b2, fw, fb, out):
    mu2 = s2[...] / _N
    var2 = q2[...] / _N - mu2 * mu2
    sc2 = g2[...] * lax.rsqrt(var2 + 1e-5)
    sh2 = b2[...] - mu2 * sc2
    x = summed[...] * sc2 + sh2
    a = af[...]
    af3 = jnp.concatenate([a, a, a], axis=1)
    o = _softplus(af3 + x)
    y = jnp.dot(o, fw[...], preferred_element_type=jnp.float32) + fb[...]
    out[...] = jnp.maximum(y, 0.0)


def _rep_spec(shape):
    return pl.BlockSpec(shape, lambda i: tuple(0 for _ in shape))


def kernel(atom_fea, nbr_fea, nbr_fea_idx, params):
    ws_l, wn_l, we_l, b_l = [], [], [], []
    scales = (1, 2, 3)
    for i, s in enumerate(scales):
        w = params["fc_W_%d" % i]
        b = params["fc_b_%d" % i]
        w_self, w_nbr, w_e = w[:_A], w[_A : 2 * _A], w[2 * _A :]
        if s > 1:
            b = b + params["st_b_%d" % s] @ w_e
            w_e = params["st_W_%d" % s] @ w_e
        ws_l.append(w_self)
        wn_l.append(w_nbr)
        we_l.append(w_e)
        b_l.append(b)
    ws_all = jnp.concatenate(ws_l, axis=1)
    wn_all = jnp.concatenate(wn_l, axis=1)
    we_all = jnp.concatenate(we_l, axis=1)
    b_all = jnp.concatenate(b_l)
    g1 = jnp.concatenate([params["bn1_g_%d" % i] for i in range(_S)])
    b1 = jnp.concatenate([params["bn1_b_%d" % i] for i in range(_S)])
    g2 = jnp.concatenate([params["bn2_g_%d" % i] for i in range(_S)])
    b2 = jnp.concatenate([params["bn2_b_%d" % i] for i in range(_S)])

    table = atom_fea.astype(jnp.bfloat16)
    idx_flat = nbr_fea_idx.reshape(-1).astype(jnp.int32)
    anbr = _sc_gather(table, idx_flat)

    e2d = nbr_fea.reshape(_R, _NB)
    e2d_bf = e2d.astype(jnp.bfloat16)

    grid1 = _N // _BN1
    nn, ne, ns, ee, es, ss, cn, ce, ca = pl.pallas_call(
        _p1_body,
        grid=(grid1,),
        in_specs=[
            pl.BlockSpec((_RB, _A), lambda i: (i, 0)),
            pl.BlockSpec((_RB, _NB), lambda i: (i, 0)),
            pl.BlockSpec((_BN1, _A), lambda i: (i, 0)),
        ],
        out_specs=[
            _rep_spec((_A, _A)),
            _rep_spec((_A, _NB)),
            _rep_spec((_A, _A)),
            _rep_spec((_NB, _NB)),
            _rep_spec((_NB, _A)),
            _rep_spec((_A, _A)),
            _rep_spec((1, _A)),
            _rep_spec((1, _NB)),
            _rep_spec((1, _A)),
        ],
        out_shape=[
            jax.ShapeDtypeStruct((_A, _A), jnp.float32),
            jax.ShapeDtypeStruct((_A, _NB), jnp.float32),
            jax.ShapeDtypeStruct((_A, _A), jnp.float32),
            jax.ShapeDtypeStruct((_NB, _NB), jnp.float32),
            jax.ShapeDtypeStruct((_NB, _A), jnp.float32),
            jax.ShapeDtypeStruct((_A, _A), jnp.float32),
            jax.ShapeDtypeStruct((1, _A), jnp.float32),
            jax.ShapeDtypeStruct((1, _NB), jnp.float32),
            jax.ShapeDtypeStruct((1, _A), jnp.float32),
        ],
    )(anbr, e2d_bf, atom_fea)

    bmat = jnp.asarray(
        np.pad(
            np.kron(np.eye(_BN1, dtype=np.float32), np.ones((_M, 1), np.float32)),
            ((0, 0), (0, _ZP)),
        ),
        dtype=jnp.bfloat16,
    )
    bmat_t = jnp.asarray(
        np.kron(np.eye(_BN1, dtype=np.float32), np.ones((1, _M), np.float32)),
        dtype=jnp.bfloat16,
    )

    summed, s2, q2 = pl.pallas_call(
        _p2_body,
        grid=(grid1,),
        in_specs=[
            pl.BlockSpec((_RB, _A), lambda i: (i, 0)),
            pl.BlockSpec((_RB, _NB), lambda i: (i, 0)),
            pl.BlockSpec((_BN1, _A), lambda i: (i, 0)),
            _rep_spec((_A, _G)),
            _rep_spec((_A, _G)),
            _rep_spec((_NB, _G)),
            _rep_spec((1, _G)),
            _rep_spec((1, _G)),
            _rep_spec((_A, _A)),
            _rep_spec((_A, _NB)),
            _rep_spec((_A, _A)),
            _rep_spec((_NB, _NB)),
            _rep_spec((_NB, _A)),
            _rep_spec((_A, _A)),
            _rep_spec((1, _A)),
            _rep_spec((1, _NB)),
            _rep_spec((1, _A)),
            _rep_spec((_RB, _BP)),
            _rep_spec((_BN1, _RB)),
        ],
        out_specs=[
            pl.BlockSpec((_BN1, _S * _A), lambda i: (i, 0)),
            _rep_spec((1, _S * _A)),
            _rep_spec((1, _S * _A)),
        ],
        out_shape=[
            jax.ShapeDtypeStruct((_N, _S * _A), jnp.float32),
            jax.ShapeDtypeStruct((1, _S * _A), jnp.float32),
            jax.ShapeDtypeStruct((1, _S * _A), jnp.float32),
        ],
        scratch_shapes=[
            pltpu.VMEM((_A, _G), jnp.bfloat16),
            pltpu.VMEM((_A, _G), jnp.float32),
            pltpu.VMEM((_NB, _G), jnp.bfloat16),
            pltpu.VMEM((1, _G), jnp.float32),
        ],
    )(
        anbr,
        e2d_bf,
        atom_fea,
        wn_all,
        ws_all,
        we_all,
        g1[None],
        b1[None],
        nn, ne, ns, ee, es, ss, cn, ce, ca,
        bmat,
        bmat_t,
    )

    out = pl.pallas_call(
        _p3_body,
        grid=(_N // _BN3,),
        in_specs=[
            pl.BlockSpec((_BN3, _S * _A), lambda i: (i, 0)),
            pl.BlockSpec((_BN3, _A), lambda i: (i, 0)),
            _rep_spec((1, _S * _A)),
            _rep_spec((1, _S * _A)),
            _rep_spec((1, _S * _A)),
            _rep_spec((1, _S * _A)),
            _rep_spec((_S * _A, _A)),
            _rep_spec((1, _A)),
        ],
        out_specs=pl.BlockSpec((_BN3, _A), lambda i: (i, 0)),
        out_shape=jax.ShapeDtypeStruct((_N, _A), jnp.float32),
    )(summed, atom_fea, s2, q2, g2[None], b2[None],
      params["fus_W"], params["fus_b"][None])
    return out
